# baseline (speedup 1.0000x reference)
import math

import jax
import jax.numpy as jnp
from jax.experimental import pallas as pl
from jax.experimental.pallas import tpu as pltpu


def _fused_body(x_ref, ctx_ref, W0_ref, b0_ref, W1_ref, b1_ref,
                Wkv_ref, bkv_ref, Wq_ref, bq_ref,
                emb_ref, w_ref, q_scr, val_scr, lg_scr):
    H = W1_ref.shape[1]
    B = ctx_ref.shape[0]
    b = pl.program_id(0)
    slot = jax.lax.rem(b, 2)

    @pl.when(b == 0)
    def _():
        q_scr[...] = jnp.dot(ctx_ref[:, 0, :], Wq_ref[...],
                             preferred_element_type=jnp.float32) + bq_ref[...]

    def tail(src_slot, row):
        lg = lg_scr[src_slot]
        m = jnp.max(lg)
        ex = jnp.exp(lg - m)
        w = ex / jnp.sum(ex)
        emb_ref[pl.ds(row, 1), :] = jnp.dot(w, val_scr[src_slot],
                                            preferred_element_type=jnp.float32)
        w_ref[pl.ds(row, 1), :] = w

    x = x_ref[0].astype(jnp.bfloat16)
    h = jnp.maximum(jnp.dot(x, W0_ref[...], preferred_element_type=jnp.float32)
                    + b0_ref[...], 0.0)

    tail(1 - slot, jnp.maximum(b - 1, 0))

    h = jnp.maximum(jnp.dot(h.astype(jnp.bfloat16), W1_ref[...],
                            preferred_element_type=jnp.float32)
                    + b1_ref[...], 0.0)
    kv = jnp.dot(h.astype(jnp.bfloat16), Wkv_ref[...],
                 preferred_element_type=jnp.float32) + bkv_ref[...]
    q = q_scr[pl.ds(b, 1), :]
    logits = jax.lax.dot_general(
        q, kv[:, :H], (((1,), (1,)), ((), ())),
        preferred_element_type=jnp.float32) * (1.0 / math.sqrt(H))
    val_scr[slot] = kv[:, H:]
    lg_scr[slot] = logits

    @pl.when(b == B - 1)
    def _():
        tail(slot, B - 1)


def kernel(objects_list, context, W0, b0, W1, b1, Wkv, bkv, Wq, bq):
    B, L, D = objects_list.shape
    D_CTX = context.shape[1]
    H = W1.shape[1]
    W0b = W0.astype(jnp.bfloat16)
    W1b = W1.astype(jnp.bfloat16)
    Wkvb = Wkv.astype(jnp.bfloat16)
    ctx3 = context.reshape(B, 1, D_CTX)
    b0r = b0.reshape(1, -1)
    b1r = b1.reshape(1, -1)
    bkvr = bkv.reshape(1, -1)
    bqr = bq.reshape(1, -1)

    full = lambda shape: pl.BlockSpec(shape, lambda b: (0,) * len(shape))
    emb, w = pl.pallas_call(
        _fused_body,
        grid=(B,),
        in_specs=[
            pl.BlockSpec((1, L, D), lambda b: (b, 0, 0)),
            full(ctx3.shape),
            full(W0b.shape), full(b0r.shape),
            full(W1b.shape), full(b1r.shape),
            full(Wkvb.shape), full(bkvr.shape),
            full(Wq.shape), full(bqr.shape),
        ],
        out_specs=[
            full((B, H)),
            full((B, L)),
        ],
        out_shape=[
            jax.ShapeDtypeStruct((B, H), jnp.float32),
            jax.ShapeDtypeStruct((B, L), jnp.float32),
        ],
        scratch_shapes=[
            pltpu.VMEM((B, H), jnp.float32),
            pltpu.VMEM((2, L, H), jnp.float32),
            pltpu.VMEM((2, 1, L), jnp.float32),
        ],
    )(objects_list, ctx3, W0b, b0r, W1b, b1r, Wkvb, bkvr, Wq, bqr)
    return emb, w


# speedup vs baseline: 1.1811x; 1.1811x over previous
import math

import jax
import jax.numpy as jnp
from jax.experimental import pallas as pl
from jax.experimental.pallas import tpu as pltpu


def _fused_body(x_ref, ctx_ref, W0_ref, b0_ref, W1_ref, b1_ref,
                Wkv_ref, bkv_ref, Wq_ref, bq_ref,
                emb_ref, w_ref, q_scr, val_scr, lg_scr):
    H = W1_ref.shape[1]
    B = ctx_ref.shape[0]
    b = pl.program_id(0)
    slot = jax.lax.rem(b, 2)

    @pl.when(b == 0)
    def _():
        q_scr[...] = jnp.dot(ctx_ref[:, 0, :], Wq_ref[...],
                             preferred_element_type=jnp.float32) + bq_ref[...]

    def tail(src_slot, row):
        lg = lg_scr[src_slot]
        m = jnp.max(lg)
        ex = jnp.exp(lg - m)
        w = ex / jnp.sum(ex)
        emb_ref[pl.ds(row, 1), :] = jnp.dot(w, val_scr[src_slot],
                                            preferred_element_type=jnp.float32)
        w_ref[pl.ds(row, 1), :] = w

    x = x_ref[0]
    h = jnp.maximum(jnp.dot(x, W0_ref[...], preferred_element_type=jnp.float32)
                    + b0_ref[...], 0.0)

    tail(1 - slot, jnp.maximum(b - 1, 0))

    h = jnp.maximum(jnp.dot(h, W1_ref[...], preferred_element_type=jnp.float32)
                    + b1_ref[...], 0.0)
    kv = jnp.dot(h, Wkv_ref[...], preferred_element_type=jnp.float32) + bkv_ref[...]
    q = q_scr[pl.ds(b, 1), :]
    logits = jax.lax.dot_general(
        q, kv[:, :H], (((1,), (1,)), ((), ())),
        preferred_element_type=jnp.float32) * (1.0 / math.sqrt(H))
    val_scr[slot] = kv[:, H:]
    lg_scr[slot] = logits

    @pl.when(b == B - 1)
    def _():
        tail(slot, B - 1)


def kernel(objects_list, context, W0, b0, W1, b1, Wkv, bkv, Wq, bq):
    B, L, D = objects_list.shape
    D_CTX = context.shape[1]
    H = W1.shape[1]
    ctx3 = context.reshape(B, 1, D_CTX)
    b0r = b0.reshape(1, -1)
    b1r = b1.reshape(1, -1)
    bkvr = bkv.reshape(1, -1)
    bqr = bq.reshape(1, -1)

    full = lambda shape: pl.BlockSpec(shape, lambda b: (0,) * len(shape))
    emb, w = pl.pallas_call(
        _fused_body,
        grid=(B,),
        in_specs=[
            pl.BlockSpec((1, L, D), lambda b: (b, 0, 0)),
            full(ctx3.shape),
            full(W0.shape), full(b0r.shape),
            full(W1.shape), full(b1r.shape),
            full(Wkv.shape), full(bkvr.shape),
            full(Wq.shape), full(bqr.shape),
        ],
        out_specs=[
            full((B, H)),
            full((B, L)),
        ],
        out_shape=[
            jax.ShapeDtypeStruct((B, H), jnp.float32),
            jax.ShapeDtypeStruct((B, L), jnp.float32),
        ],
        scratch_shapes=[
            pltpu.VMEM((B, H), jnp.float32),
            pltpu.VMEM((2, L, H), jnp.float32),
            pltpu.VMEM((2, 1, L), jnp.float32),
        ],
    )(objects_list, ctx3, W0, b0r, W1, b1r, Wkv, bkvr, Wq, bqr)
    return emb, w
